# initial kernel scaffold (unmeasured)
import jax
import jax.numpy as jnp
from jax import lax
from jax.experimental import pallas as pl
from jax.experimental.pallas import tpu as pltpu

N_DEV = 16
M = 1024
N = 1024
CH = M // N_DEV

_GELU_C = 0.7978845608028654


def kernel(x, w_mat):
    def body(x_ref, w_ref, out_ref, send_buf, rs_buf, ag_src, ag_buf,
             rs_send_sems, rs_recv_sems, ag_send_sems, ag_recv_sems):
        me = lax.axis_index("i")

        for d in range(N_DEV):
            part = jnp.dot(x_ref[d * CH:(d + 1) * CH, :], w_ref[...],
                           preferred_element_type=jnp.float32)
            send_buf[d] = part.astype(jnp.bfloat16)

        rs_sends = []
        for off in range(1, N_DEV):
            dst = (me + off) % N_DEV
            rdma = pltpu.make_async_remote_copy(
                src_ref=send_buf.at[dst],
                dst_ref=rs_buf.at[me],
                send_sem=rs_send_sems.at[dst],
                recv_sem=rs_recv_sems.at[me],
                device_id=(dst,),
                device_id_type=pl.DeviceIdType.MESH,
            )
            rdma.start()
            rs_sends.append(rdma)

        own = pl.load(send_buf, (pl.ds(me, 1), slice(None), slice(None)))
        pl.store(rs_buf, (pl.ds(me, 1), slice(None), slice(None)), own)

        for off in range(1, N_DEV):
            src = (me + N_DEV - off) % N_DEV
            recv = pltpu.make_async_remote_copy(
                src_ref=send_buf.at[0],
                dst_ref=rs_buf.at[src],
                send_sem=rs_send_sems.at[0],
                recv_sem=rs_recv_sems.at[src],
                device_id=(me,),
                device_id_type=pl.DeviceIdType.MESH,
            )
            recv.wait_recv()

        acc = jnp.sum(rs_buf[...].astype(jnp.float32), axis=0)
        g = 0.5 * acc * (1.0 + jnp.tanh(_GELU_C * (acc + 0.044715 * acc ** 3)))
        pl.store(out_ref, (pl.ds(me * CH, CH), slice(None)), g)
        ag_src[...] = g.astype(jnp.bfloat16)

        for r in rs_sends:
            r.wait_send()

        ag_sends = []
        for off in range(1, N_DEV):
            dst = (me + off) % N_DEV
            rdma = pltpu.make_async_remote_copy(
                src_ref=ag_src,
                dst_ref=ag_buf.at[me],
                send_sem=ag_send_sems.at[dst],
                recv_sem=ag_recv_sems.at[me],
                device_id=(dst,),
                device_id_type=pl.DeviceIdType.MESH,
            )
            rdma.start()
            ag_sends.append(rdma)

        for off in range(1, N_DEV):
            src = (me + N_DEV - off) % N_DEV
            recv = pltpu.make_async_remote_copy(
                src_ref=ag_src,
                dst_ref=ag_buf.at[src],
                send_sem=ag_send_sems.at[0],
                recv_sem=ag_recv_sems.at[src],
                device_id=(me,),
                device_id_type=pl.DeviceIdType.MESH,
            )
            recv.wait_recv()
            chunk = pl.load(ag_buf, (pl.ds(src, 1), slice(None), slice(None)))
            pl.store(out_ref, (pl.ds(src * CH, CH), slice(None)),
                     chunk[0].astype(jnp.float32))

        for r in ag_sends:
            r.wait_send()

    return pl.pallas_call(
        body,
        out_shape=jax.ShapeDtypeStruct((M, N), jnp.float32),
        in_specs=[pl.BlockSpec(memory_space=pltpu.VMEM),
                  pl.BlockSpec(memory_space=pltpu.VMEM)],
        out_specs=pl.BlockSpec(memory_space=pltpu.VMEM),
        scratch_shapes=[
            pltpu.VMEM((N_DEV, CH, N), jnp.bfloat16),
            pltpu.VMEM((N_DEV, CH, N), jnp.bfloat16),
            pltpu.VMEM((CH, N), jnp.bfloat16),
            pltpu.VMEM((N_DEV, CH, N), jnp.bfloat16),
            pltpu.SemaphoreType.DMA((N_DEV,)),
            pltpu.SemaphoreType.DMA((N_DEV,)),
            pltpu.SemaphoreType.DMA((N_DEV,)),
            pltpu.SemaphoreType.DMA((N_DEV,)),
        ],
        compiler_params=pltpu.CompilerParams(collective_id=0),
    )(x, w_mat)


# baseline (device time: 58294 ns/iter reference)
import jax
import jax.numpy as jnp
from jax import lax
from jax.experimental import pallas as pl
from jax.experimental.pallas import tpu as pltpu

N_DEV = 16
M = 1024
N = 1024
CH = M // N_DEV

_GELU_C = 0.7978845608028654


def kernel(x, w_mat):
    def body(x_ref, w_ref, out_ref, send_buf, rs_buf, ag_src, ag_buf,
             rs_send_sems, rs_recv_sems, ag_send_sems, ag_recv_sems):
        me = lax.axis_index("i")

        for d in range(N_DEV):
            part = jnp.dot(x_ref[d * CH:(d + 1) * CH, :], w_ref[...],
                           preferred_element_type=jnp.float32)
            send_buf[d] = part.astype(jnp.bfloat16)

        rs_sends = []
        for off in range(1, N_DEV):
            dst = (me + off) % N_DEV
            rdma = pltpu.make_async_remote_copy(
                src_ref=send_buf.at[dst],
                dst_ref=rs_buf.at[me],
                send_sem=rs_send_sems.at[dst],
                recv_sem=rs_recv_sems.at[me],
                device_id=(dst,),
                device_id_type=pl.DeviceIdType.MESH,
            )
            rdma.start()
            rs_sends.append(rdma)

        rs_buf[pl.ds(me, 1), :, :] = send_buf[pl.ds(me, 1), :, :]

        for off in range(1, N_DEV):
            src = (me + N_DEV - off) % N_DEV
            recv = pltpu.make_async_remote_copy(
                src_ref=send_buf.at[0],
                dst_ref=rs_buf.at[src],
                send_sem=rs_send_sems.at[0],
                recv_sem=rs_recv_sems.at[src],
                device_id=(me,),
                device_id_type=pl.DeviceIdType.MESH,
            )
            recv.wait_recv()

        acc = jnp.sum(rs_buf[...].astype(jnp.float32), axis=0)
        g = 0.5 * acc * (1.0 + jnp.tanh(_GELU_C * (acc + 0.044715 * acc ** 3)))
        out_ref[pl.ds(me * CH, CH), :] = g
        ag_src[...] = g.astype(jnp.bfloat16)

        for r in rs_sends:
            r.wait_send()

        ag_sends = []
        for off in range(1, N_DEV):
            dst = (me + off) % N_DEV
            rdma = pltpu.make_async_remote_copy(
                src_ref=ag_src,
                dst_ref=ag_buf.at[me],
                send_sem=ag_send_sems.at[dst],
                recv_sem=ag_recv_sems.at[me],
                device_id=(dst,),
                device_id_type=pl.DeviceIdType.MESH,
            )
            rdma.start()
            ag_sends.append(rdma)

        for off in range(1, N_DEV):
            src = (me + N_DEV - off) % N_DEV
            recv = pltpu.make_async_remote_copy(
                src_ref=ag_src,
                dst_ref=ag_buf.at[src],
                send_sem=ag_send_sems.at[0],
                recv_sem=ag_recv_sems.at[src],
                device_id=(me,),
                device_id_type=pl.DeviceIdType.MESH,
            )
            recv.wait_recv()
            chunk = ag_buf[pl.ds(src, 1), :, :]
            out_ref[pl.ds(src * CH, CH), :] = chunk[0].astype(jnp.float32)

        for r in ag_sends:
            r.wait_send()

    return pl.pallas_call(
        body,
        out_shape=jax.ShapeDtypeStruct((M, N), jnp.float32),
        in_specs=[pl.BlockSpec(memory_space=pltpu.VMEM),
                  pl.BlockSpec(memory_space=pltpu.VMEM)],
        out_specs=pl.BlockSpec(memory_space=pltpu.VMEM),
        scratch_shapes=[
            pltpu.VMEM((N_DEV, CH, N), jnp.bfloat16),
            pltpu.VMEM((N_DEV, CH, N), jnp.bfloat16),
            pltpu.VMEM((CH, N), jnp.bfloat16),
            pltpu.VMEM((N_DEV, CH, N), jnp.bfloat16),
            pltpu.SemaphoreType.DMA((N_DEV,)),
            pltpu.SemaphoreType.DMA((N_DEV,)),
            pltpu.SemaphoreType.DMA((N_DEV,)),
            pltpu.SemaphoreType.DMA((N_DEV,)),
        ],
    )(x, w_mat)


# device time: 48589 ns/iter; 1.1997x vs baseline; 1.1997x over previous
import jax
import jax.numpy as jnp
from jax import lax
from jax.experimental import pallas as pl
from jax.experimental.pallas import tpu as pltpu

N_DEV = 16
M = 1024
N = 1024
CH = M // N_DEV
IH = 8

_GELU_C = 0.7978845608028654


def kernel(x, w_mat):
    def body(x_ref, w_ref, out_ref, send_buf, pA, pB, sb, fs, ag_src,
             mir_buf, agA, agB,
             p1a_send, p1b_send, pA_recv, pB_recv,
             p2_send, fs_recv, mir_send, mir_recv,
             agA_send, agA_recv, agB_send, agB_recv):
        me = lax.axis_index("i")
        z = me // 4
        p = me % 4
        h = z // 2
        my_ih = (z % 2) * 4 + p
        mir_me = 4 * (3 - z) + p

        def ih_to_idx(ihh):
            return 4 * (2 * h + ihh // 4) + ihh % 4

        def mirror_of(idx):
            return 4 * (3 - idx // 4) + idx % 4

        for d in range(N_DEV):
            part = jnp.dot(x_ref[d * CH:(d + 1) * CH, :], w_ref[...],
                           preferred_element_type=jnp.float32)
            send_buf[d] = part.astype(jnp.bfloat16)

        p1_sends = []
        for off in range(1, IH):
            peer_ih = (my_ih + off) % IH
            peer_idx = ih_to_idx(peer_ih)
            peer_mir = mirror_of(peer_idx)
            ra = pltpu.make_async_remote_copy(
                src_ref=send_buf.at[peer_idx], dst_ref=pA.at[my_ih],
                send_sem=p1a_send.at[off], recv_sem=pA_recv.at[my_ih],
                device_id=(peer_idx,), device_id_type=pl.DeviceIdType.MESH)
            rb = pltpu.make_async_remote_copy(
                src_ref=send_buf.at[peer_mir], dst_ref=pB.at[my_ih],
                send_sem=p1b_send.at[off], recv_sem=pB_recv.at[my_ih],
                device_id=(peer_idx,), device_id_type=pl.DeviceIdType.MESH)
            ra.start()
            rb.start()
            p1_sends += [ra, rb]

        pA[pl.ds(my_ih, 1), :, :] = send_buf[pl.ds(me, 1), :, :]
        pB[pl.ds(my_ih, 1), :, :] = send_buf[pl.ds(mir_me, 1), :, :]

        for off in range(1, IH):
            src_ih = (my_ih + IH - off) % IH
            for buf, rsem, ssem in ((pA, pA_recv, p1a_send),
                                    (pB, pB_recv, p1b_send)):
                pltpu.make_async_remote_copy(
                    src_ref=send_buf.at[0], dst_ref=buf.at[src_ih],
                    send_sem=ssem.at[0], recv_sem=rsem.at[src_ih],
                    device_id=(me,), device_id_type=pl.DeviceIdType.MESH,
                ).wait_recv()

        s_a = jnp.sum(pA[...].astype(jnp.float32), axis=0)
        s_b = jnp.sum(pB[...].astype(jnp.float32), axis=0)
        sb[...] = s_b.astype(jnp.bfloat16)

        p2 = pltpu.make_async_remote_copy(
            src_ref=sb, dst_ref=fs,
            send_sem=p2_send.at[0], recv_sem=fs_recv.at[0],
            device_id=(mir_me,), device_id_type=pl.DeviceIdType.MESH)
        p2.start()
        pltpu.make_async_remote_copy(
            src_ref=sb, dst_ref=fs, send_sem=p2_send.at[0],
            recv_sem=fs_recv.at[0], device_id=(me,),
            device_id_type=pl.DeviceIdType.MESH).wait_recv()

        total = s_a + fs[...].astype(jnp.float32)
        g = 0.5 * total * (1.0 + jnp.tanh(_GELU_C * (total + 0.044715 * total ** 3)))
        out_ref[pl.ds(me * CH, CH), :] = g
        ag_src[...] = g.astype(jnp.bfloat16)

        mir = pltpu.make_async_remote_copy(
            src_ref=ag_src, dst_ref=mir_buf,
            send_sem=mir_send.at[0], recv_sem=mir_recv.at[0],
            device_id=(mir_me,), device_id_type=pl.DeviceIdType.MESH)
        mir.start()
        ag_sends = [mir]
        for off in range(1, IH):
            peer_ih = (my_ih + off) % IH
            peer_idx = ih_to_idx(peer_ih)
            rdma = pltpu.make_async_remote_copy(
                src_ref=ag_src, dst_ref=agA.at[my_ih],
                send_sem=agA_send.at[off], recv_sem=agA_recv.at[my_ih],
                device_id=(peer_idx,), device_id_type=pl.DeviceIdType.MESH)
            rdma.start()
            ag_sends.append(rdma)

        pltpu.make_async_remote_copy(
            src_ref=ag_src, dst_ref=mir_buf, send_sem=mir_send.at[0],
            recv_sem=mir_recv.at[0], device_id=(me,),
            device_id_type=pl.DeviceIdType.MESH).wait_recv()
        out_ref[pl.ds(mir_me * CH, CH), :] = mir_buf[...].astype(jnp.float32)
        for off in range(1, IH):
            peer_ih = (my_ih + off) % IH
            peer_idx = ih_to_idx(peer_ih)
            rdma = pltpu.make_async_remote_copy(
                src_ref=mir_buf, dst_ref=agB.at[my_ih],
                send_sem=agB_send.at[off], recv_sem=agB_recv.at[my_ih],
                device_id=(peer_idx,), device_id_type=pl.DeviceIdType.MESH)
            rdma.start()
            ag_sends.append(rdma)

        for off in range(1, IH):
            src_ih = (my_ih + IH - off) % IH
            src_idx = ih_to_idx(src_ih)
            pltpu.make_async_remote_copy(
                src_ref=ag_src, dst_ref=agA.at[src_ih],
                send_sem=agA_send.at[0], recv_sem=agA_recv.at[src_ih],
                device_id=(me,), device_id_type=pl.DeviceIdType.MESH,
            ).wait_recv()
            out_ref[pl.ds(src_idx * CH, CH), :] = (
                agA[pl.ds(src_ih, 1), :, :][0].astype(jnp.float32))
        for off in range(1, IH):
            src_ih = (my_ih + IH - off) % IH
            far_idx = mirror_of(ih_to_idx(src_ih))
            pltpu.make_async_remote_copy(
                src_ref=ag_src, dst_ref=agB.at[src_ih],
                send_sem=agB_send.at[0], recv_sem=agB_recv.at[src_ih],
                device_id=(me,), device_id_type=pl.DeviceIdType.MESH,
            ).wait_recv()
            out_ref[pl.ds(far_idx * CH, CH), :] = (
                agB[pl.ds(src_ih, 1), :, :][0].astype(jnp.float32))

        for r in p1_sends + [p2] + ag_sends:
            r.wait_send()

    return pl.pallas_call(
        body,
        out_shape=jax.ShapeDtypeStruct((M, N), jnp.float32),
        in_specs=[pl.BlockSpec(memory_space=pltpu.VMEM),
                  pl.BlockSpec(memory_space=pltpu.VMEM)],
        out_specs=pl.BlockSpec(memory_space=pltpu.VMEM),
        scratch_shapes=[
            pltpu.VMEM((N_DEV, CH, N), jnp.bfloat16),
            pltpu.VMEM((IH, CH, N), jnp.bfloat16),
            pltpu.VMEM((IH, CH, N), jnp.bfloat16),
            pltpu.VMEM((CH, N), jnp.bfloat16),
            pltpu.VMEM((CH, N), jnp.bfloat16),
            pltpu.VMEM((CH, N), jnp.bfloat16),
            pltpu.VMEM((CH, N), jnp.bfloat16),
            pltpu.VMEM((IH, CH, N), jnp.bfloat16),
            pltpu.VMEM((IH, CH, N), jnp.bfloat16),
            pltpu.SemaphoreType.DMA((IH,)),
            pltpu.SemaphoreType.DMA((IH,)),
            pltpu.SemaphoreType.DMA((IH,)),
            pltpu.SemaphoreType.DMA((IH,)),
            pltpu.SemaphoreType.DMA((1,)),
            pltpu.SemaphoreType.DMA((1,)),
            pltpu.SemaphoreType.DMA((1,)),
            pltpu.SemaphoreType.DMA((1,)),
            pltpu.SemaphoreType.DMA((IH,)),
            pltpu.SemaphoreType.DMA((IH,)),
            pltpu.SemaphoreType.DMA((IH,)),
            pltpu.SemaphoreType.DMA((IH,)),
        ],
    )(x, w_mat)
